# Initial kernel scaffold; baseline (speedup 1.0000x reference)
import jax
import jax.numpy as jnp
from jax.experimental import pallas as pl


def kernel(input, rois, offsets):
    raise NotImplementedError("write your pallas kernel here")


# speedup vs baseline: 15.3016x; 15.3016x over previous
import functools

import jax
import jax.numpy as jnp
from jax import lax
from jax.experimental import pallas as pl
from jax.experimental.pallas import tpu as pltpu
from jax.experimental.pallas import tpu_sc as plsc

_PH, _PW = 7, 7
_S = 2
_SCALE = 0.125
_TRANS_STD = 0.1
_NBINS = _PH * _PW
_K = _S * _S * 4
_J = _NBINS * _K


def _stage_a_body(R, H, W, RB, roi_ref, offx_ref, offy_ref, idx_ref, w_ref):
    f32 = jnp.float32
    j = lax.broadcasted_iota(jnp.int32, (RB, _J), 1)
    r_glob = lax.broadcasted_iota(jnp.int32, (RB, _J), 0) + pl.program_id(0) * RB
    corner = j & 3
    sx = ((j >> 2) & 1).astype(f32)
    sy = ((j >> 3) & 1).astype(f32)
    binf = (j >> 4).astype(f32)
    ph = jnp.floor((binf + 0.5) * (1.0 / _PW))
    pw = binf - ph * float(_PW)

    b = roi_ref[:, 0:1]
    x1 = roi_ref[:, 1:2]
    y1 = roi_ref[:, 2:3]
    x2 = roi_ref[:, 3:4]
    y2 = roi_ref[:, 4:5]
    rsw = x1 * _SCALE - 0.5
    rsh = y1 * _SCALE - 0.5
    rew = x2 * _SCALE - 0.5
    reh = y2 * _SCALE - 0.5
    roi_w = rew - rsw
    roi_h = reh - rsh
    bin_w = roi_w / _PW
    bin_h = roi_h / _PH
    dh = offy_ref[...] * _TRANS_STD * roi_h
    dw = offx_ref[...] * _TRANS_STD * roi_w
    ys = rsh + ph * bin_h + (sy + 0.5) * bin_h / _S + dh
    xs = rsw + pw * bin_w + (sx + 0.5) * bin_w / _S + dw

    valid = (ys >= -1.0) & (ys <= float(H)) & (xs >= -1.0) & (xs <= float(W))
    y = jnp.maximum(ys, 0.0)
    x = jnp.maximum(xs, 0.0)
    y_low = jnp.minimum(jnp.floor(y), float(H - 1))
    x_low = jnp.minimum(jnp.floor(x), float(W - 1))
    y_high = jnp.minimum(y_low + 1.0, float(H - 1))
    x_high = jnp.minimum(x_low + 1.0, float(W - 1))
    y = jnp.where(y_low >= float(H - 1), jnp.float32(H - 1), y)
    x = jnp.where(x_low >= float(W - 1), jnp.float32(W - 1), x)
    ly = y - y_low
    lx = x - x_low
    hy = 1.0 - ly
    hx = 1.0 - lx

    use_yh = corner >= 2
    use_xh = (corner & 1) == 1
    yy = jnp.where(use_yh, y_high, y_low)
    xx = jnp.where(use_xh, x_high, x_low)
    wy = jnp.where(use_yh, ly, hy)
    wx = jnp.where(use_xh, lx, hx)
    wgt = wy * wx * 0.25
    wgt = jnp.where(valid & (r_glob < R), wgt, 0.0)

    idx = b * float(H * W) + yy * float(W) + xx
    idx_ref[...] = jnp.clip(idx.astype(jnp.int32), 0, H * W * 2 - 1)
    w_ref[...] = wgt


def _make_stage_a(R, RPAD, H, W):
    RB = 128
    body = functools.partial(_stage_a_body, R, H, W, RB)
    spec = pl.BlockSpec((RB, _J), lambda i: (i, 0))
    return pl.pallas_call(
        body,
        grid=(RPAD // RB,),
        in_specs=[pl.BlockSpec((RB, 5), lambda i: (i, 0)), spec, spec],
        out_specs=(spec, spec),
        out_shape=(jax.ShapeDtypeStruct((RPAD, _J), jnp.int32),
                   jax.ShapeDtypeStruct((RPAD, _J), jnp.float32)),
    )


def _splat(vec, k):
    dn = lax.GatherDimensionNumbers(
        offset_dims=(), collapsed_slice_dims=(0,), start_index_map=(0,))
    idx = jnp.full((16, 1), k, jnp.int32)
    return lax.gather(vec, idx, dn, (1,),
                      mode=lax.GatherScatterMode.PROMISE_IN_BOUNDS)


def _make_stage_b(NB, C):
    NW = 32
    per_w = NB // NW
    SB = 8
    ROWS = SB * _K
    n_sub = per_w // SB
    assert NB % NW == 0 and per_w % SB == 0 and n_sub % 2 == 0
    mesh = plsc.VectorSubcoreMesh(core_axis_name="c", subcore_axis_name="s")

    @functools.partial(
        pl.kernel, mesh=mesh,
        out_type=jax.ShapeDtypeStruct((NB, C), jnp.float32),
        scratch_types=[
            pltpu.VMEM((per_w * _K,), jnp.int32),
            pltpu.VMEM((per_w * _K,), jnp.float32),
            pltpu.VMEM((ROWS, C), jnp.float32),
            pltpu.VMEM((ROWS, C), jnp.float32),
            pltpu.VMEM((SB, C), jnp.float32),
            pltpu.SemaphoreType.DMA,
            pltpu.SemaphoreType.DMA,
        ])
    def sc(table, idx_h, w_h, out_h, idx_v, w_v, buf0, buf1, out_v, sem0, sem1):
        wid = lax.axis_index("s") * 2 + lax.axis_index("c")
        base = wid * (per_w * _K)
        pltpu.sync_copy(idx_h.at[pl.ds(base, per_w * _K)], idx_v)
        pltpu.sync_copy(w_h.at[pl.ds(base, per_w * _K)], w_v)
        bufs = (buf0, buf1)
        sems = (sem0, sem1)

        def start(sb, b):
            off = pl.multiple_of(sb * ROWS, ROWS)
            src = table.at[idx_v.at[pl.ds(off, ROWS)]]
            pltpu.make_async_copy(src, bufs[b], sems[b]).start()

        def wait(b):
            pltpu.make_async_copy(
                table.at[idx_v.at[pl.ds(0, ROWS)]], bufs[b], sems[b]).wait()

        start(0, 0)
        start(1, 1)

        def g_body(g, carry):
            for b in range(2):
                sb = g * 2 + b
                wait(b)
                buf = bufs[b]

                def bin_body(jj, c):
                    woff = pl.multiple_of((sb * SB + jj) * _K, _K)
                    wv = w_v[pl.ds(woff, 16)]
                    sp = [_splat(wv, k) for k in range(_K)]
                    row0 = jj * _K
                    for jc in range(C // 16):
                        cs = pl.ds(jc * 16, 16)
                        acc = sp[0] * buf[row0, cs]
                        for k in range(1, _K):
                            acc = acc + sp[k] * buf[row0 + k, cs]
                        out_v[jj, cs] = acc
                    return c

                lax.fori_loop(0, SB, bin_body, 0)
                pltpu.sync_copy(out_v, out_h.at[pl.ds(wid * per_w + sb * SB, SB)])

                @pl.when(sb + 2 < n_sub)
                def _():
                    start(sb + 2, b)
            return carry

        lax.fori_loop(0, n_sub // 2, g_body, 0)

    return sc


def kernel(input, rois, offsets):
    N, C, H, W = input.shape
    R = rois.shape[0]
    RPAD = ((R + 255) // 256) * 256

    table = jnp.transpose(input, (0, 2, 3, 1)).reshape(N * H * W, C)
    rois_p = jnp.zeros((RPAD, 5), jnp.float32).at[:R].set(rois)
    offx = jnp.zeros((RPAD, _NBINS), jnp.float32).at[:R].set(
        offsets[:, 0].reshape(R, _NBINS))
    offy = jnp.zeros((RPAD, _NBINS), jnp.float32).at[:R].set(
        offsets[:, 1].reshape(R, _NBINS))
    offx = jnp.repeat(offx, _K, axis=1)
    offy = jnp.repeat(offy, _K, axis=1)

    idx, w = _make_stage_a(R, RPAD, H, W)(rois_p, offx, offy)
    out = _make_stage_b(RPAD * _NBINS, C)(table, idx.reshape(-1), w.reshape(-1))
    out = out[: R * _NBINS].reshape(R, _NBINS, C)
    return jnp.transpose(out, (0, 2, 1)).reshape(R, C, _PH, _PW)


# speedup vs baseline: 16.2273x; 1.0605x over previous
import functools

import jax
import jax.numpy as jnp
from jax import lax
from jax.experimental import pallas as pl
from jax.experimental.pallas import tpu as pltpu
from jax.experimental.pallas import tpu_sc as plsc

_PH, _PW = 7, 7
_S = 2
_SCALE = 0.125
_TRANS_STD = 0.1
_NBINS = _PH * _PW
_K = _S * _S * 4
_J = _NBINS * _K


def _stage_a_body(R, H, W, RB, roi_ref, offx_ref, offy_ref, idx_ref, w_ref):
    f32 = jnp.float32
    j = lax.broadcasted_iota(jnp.int32, (RB, _J), 1)
    r_glob = lax.broadcasted_iota(jnp.int32, (RB, _J), 0) + pl.program_id(0) * RB
    corner = j & 3
    sx = ((j >> 2) & 1).astype(f32)
    sy = ((j >> 3) & 1).astype(f32)
    binf = (j >> 4).astype(f32)
    ph = jnp.floor((binf + 0.5) * (1.0 / _PW))
    pw = binf - ph * float(_PW)

    b = roi_ref[:, 0:1]
    x1 = roi_ref[:, 1:2]
    y1 = roi_ref[:, 2:3]
    x2 = roi_ref[:, 3:4]
    y2 = roi_ref[:, 4:5]
    rsw = x1 * _SCALE - 0.5
    rsh = y1 * _SCALE - 0.5
    rew = x2 * _SCALE - 0.5
    reh = y2 * _SCALE - 0.5
    roi_w = rew - rsw
    roi_h = reh - rsh
    bin_w = roi_w / _PW
    bin_h = roi_h / _PH
    dh = offy_ref[...] * _TRANS_STD * roi_h
    dw = offx_ref[...] * _TRANS_STD * roi_w
    ys = rsh + ph * bin_h + (sy + 0.5) * bin_h / _S + dh
    xs = rsw + pw * bin_w + (sx + 0.5) * bin_w / _S + dw

    valid = (ys >= -1.0) & (ys <= float(H)) & (xs >= -1.0) & (xs <= float(W))
    y = jnp.maximum(ys, 0.0)
    x = jnp.maximum(xs, 0.0)
    y_low = jnp.minimum(jnp.floor(y), float(H - 1))
    x_low = jnp.minimum(jnp.floor(x), float(W - 1))
    y_high = jnp.minimum(y_low + 1.0, float(H - 1))
    x_high = jnp.minimum(x_low + 1.0, float(W - 1))
    y = jnp.where(y_low >= float(H - 1), jnp.float32(H - 1), y)
    x = jnp.where(x_low >= float(W - 1), jnp.float32(W - 1), x)
    ly = y - y_low
    lx = x - x_low
    hy = 1.0 - ly
    hx = 1.0 - lx

    use_yh = corner >= 2
    use_xh = (corner & 1) == 1
    yy = jnp.where(use_yh, y_high, y_low)
    xx = jnp.where(use_xh, x_high, x_low)
    wy = jnp.where(use_yh, ly, hy)
    wx = jnp.where(use_xh, lx, hx)
    wgt = wy * wx * 0.25
    wgt = jnp.where(valid & (r_glob < R), wgt, 0.0)

    idx = b * float(H * W) + yy * float(W) + xx
    idx_ref[...] = jnp.clip(idx.astype(jnp.int32), 0, H * W * 2 - 1)
    w_ref[...] = wgt


def _make_stage_a(R, RPAD, H, W):
    RB = 128
    body = functools.partial(_stage_a_body, R, H, W, RB)
    spec = pl.BlockSpec((RB, _J), lambda i: (i, 0))
    return pl.pallas_call(
        body,
        grid=(RPAD // RB,),
        in_specs=[pl.BlockSpec((RB, 5), lambda i: (i, 0)), spec, spec],
        out_specs=(spec, spec),
        out_shape=(jax.ShapeDtypeStruct((RPAD, _J), jnp.int32),
                   jax.ShapeDtypeStruct((RPAD, _J), jnp.float32)),
    )


def _splat(vec, k):
    dn = lax.GatherDimensionNumbers(
        offset_dims=(), collapsed_slice_dims=(0,), start_index_map=(0,))
    idx = jnp.full((16, 1), k, jnp.int32)
    return lax.gather(vec, idx, dn, (1,),
                      mode=lax.GatherScatterMode.PROMISE_IN_BOUNDS)


def _make_stage_b(NB, C):
    NW = 32
    per_w = NB // NW
    SB = 8
    ROWS = SB * _K
    n_sub = per_w // SB
    assert NB % NW == 0 and per_w % SB == 0 and n_sub % 2 == 0
    mesh = plsc.VectorSubcoreMesh(core_axis_name="c", subcore_axis_name="s")

    @functools.partial(
        pl.kernel, mesh=mesh,
        out_type=jax.ShapeDtypeStruct((NB, C), jnp.float32),
        scratch_types=[
            pltpu.VMEM((per_w * _K,), jnp.int32),
            pltpu.VMEM((per_w * _K,), jnp.float32),
            pltpu.VMEM((ROWS, C), jnp.float32),
            pltpu.VMEM((ROWS, C), jnp.float32),
            pltpu.VMEM((SB, C), jnp.float32),
            pltpu.SemaphoreType.DMA,
            pltpu.SemaphoreType.DMA,
        ])
    def sc(table, idx_h, w_h, out_h, idx_v, w_v, buf0, buf1, out_v, sem0, sem1):
        wid = lax.axis_index("s") * 2 + lax.axis_index("c")
        base = wid * (per_w * _K)
        pltpu.sync_copy(idx_h.at[pl.ds(base, per_w * _K)], idx_v)
        pltpu.sync_copy(w_h.at[pl.ds(base, per_w * _K)], w_v)
        bufs = (buf0, buf1)
        sems = (sem0, sem1)

        def start(sb, b):
            off = pl.multiple_of(sb * ROWS, ROWS)
            src = table.at[idx_v.at[pl.ds(off, ROWS)]]
            pltpu.make_async_copy(src, bufs[b], sems[b]).start()

        def wait(b):
            pltpu.make_async_copy(
                table.at[idx_v.at[pl.ds(0, ROWS)]], bufs[b], sems[b]).wait()

        start(0, 0)
        start(1, 1)

        def g_body(g, carry):
            for b in range(2):
                sb = g * 2 + b
                wait(b)
                buf = bufs[b]

                def bin_body(jj):
                    woff = pl.multiple_of((sb * SB + jj) * _K, _K)
                    wv = w_v[pl.ds(woff, 16)]
                    sp = [_splat(wv, k) for k in range(_K)]
                    row0 = jj * _K

                    def ch_body(jc):
                        cs = pl.ds(pl.multiple_of(jc * 16, 16), 16)
                        prods = [sp[k] * buf[row0 + k, cs] for k in range(_K)]
                        while len(prods) > 1:
                            prods = [p + q for p, q in zip(prods[::2], prods[1::2])]
                        out_v[jj, cs] = prods[0]

                    plsc.parallel_loop(0, C // 16)(ch_body)

                plsc.parallel_loop(0, SB)(bin_body)
                pltpu.sync_copy(out_v, out_h.at[pl.ds(wid * per_w + sb * SB, SB)])

                @pl.when(sb + 2 < n_sub)
                def _():
                    start(sb + 2, b)
            return carry

        lax.fori_loop(0, n_sub // 2, g_body, 0)

    return sc


def kernel(input, rois, offsets):
    N, C, H, W = input.shape
    R = rois.shape[0]
    RPAD = ((R + 255) // 256) * 256

    table = jnp.transpose(input, (0, 2, 3, 1)).reshape(N * H * W, C)
    rois_p = jnp.zeros((RPAD, 5), jnp.float32).at[:R].set(rois)
    offx = jnp.zeros((RPAD, _NBINS), jnp.float32).at[:R].set(
        offsets[:, 0].reshape(R, _NBINS))
    offy = jnp.zeros((RPAD, _NBINS), jnp.float32).at[:R].set(
        offsets[:, 1].reshape(R, _NBINS))
    offx = jnp.repeat(offx, _K, axis=1)
    offy = jnp.repeat(offy, _K, axis=1)

    idx, w = _make_stage_a(R, RPAD, H, W)(rois_p, offx, offy)
    out = _make_stage_b(RPAD * _NBINS, C)(table, idx.reshape(-1), w.reshape(-1))
    out = out[: R * _NBINS].reshape(R, _NBINS, C)
    return jnp.transpose(out, (0, 2, 1)).reshape(R, C, _PH, _PW)
